# baseline (speedup 1.0000x reference)
import functools

import jax
import jax.numpy as jnp
from jax import lax
from jax.experimental import pallas as pl
from jax.experimental.pallas import tpu as pltpu
from jax.experimental.pallas import tpu_sc as plsc

N = 10000
E = 320000
H = 128
HH = H // 2
NGRAPHS = 64

NPAD = 10240
NC, NS = 2, 16
CH = 128
NCHUNK = 160
CHD = 128
NCHUNKD = 160
EPAD = NS * NCHUNK * CH
ROWS_PER_TILE = NPAD // NS
BLK = 1280
GRID = NPAD // BLK

_mesh = plsc.VectorSubcoreMesh(core_axis_name="c", subcore_axis_name="s")


def _zero_rows(ref, nrows, ncols):
    zero16 = jnp.zeros((16,), jnp.float32)

    def body(i):
        for k in range(ncols // 16):
            ref[i, pl.ds(k * 16, 16)] = zero16

    pl.loop(0, nrows)(body)


_DEG_KWARGS = dict(
    out_type=jax.ShapeDtypeStruct((NC, NPAD, 16), jnp.float32),
    mesh=_mesh,
    scratch_types=[
        pltpu.VMEM((NCHUNKD // 2, CHD), jnp.int32),
        pltpu.VMEM((CHD, 16), jnp.float32),
        pltpu.VMEM((CHD, 16), jnp.float32),
        pltpu.VMEM_SHARED((NPAD, 16), jnp.float32),
        pltpu.SemaphoreType.DMA,
    ],
    compiler_params=pltpu.CompilerParams(use_tc_tiling_on_sc=False),
)


def _deg_body(dst_hbm, out_hbm, idxd, ones_r, zb, acc, sem):
    c = lax.axis_index("c")
    s = lax.axis_index("s")

    pltpu.sync_copy(dst_hbm.at[s, pl.ds(c * (NCHUNKD // 2), NCHUNKD // 2)], idxd)

    one0 = jnp.where(lax.iota(jnp.int32, 16) == 0, 1.0, 0.0).astype(jnp.float32)
    zero16 = jnp.zeros((16,), jnp.float32)

    def init(i):
        ones_r[i] = one0
        zb[i] = zero16

    pl.loop(0, CHD)(init)

    base = s * ROWS_PER_TILE
    for k in range(ROWS_PER_TILE // CHD):
        pltpu.sync_copy(zb, acc.at[pl.ds(base + k * CHD, CHD)])
    plsc.subcore_barrier()

    def fire(j):
        descs = [
            pltpu.async_copy(ones_r, acc.at[idxd.at[j + k]], sem, add=True)
            for k in range(8)
        ]
        for d in descs:
            d.wait()

    pl.loop(0, NCHUNKD // 2, step=8)(fire)
    plsc.subcore_barrier()

    for k in range(ROWS_PER_TILE // CHD):
        pltpu.sync_copy(acc.at[pl.ds(base + k * CHD, CHD)], zb)
        pltpu.sync_copy(zb, out_hbm.at[c, pl.ds(base + k * CHD, CHD)])


_deg_kernel = pl.kernel(_deg_body, **_DEG_KWARGS)


NBKT = 640
EPT = NCHUNK * CH
EPL = EPT // 16

_SORT_KWARGS = dict(
    out_type=[
        jax.ShapeDtypeStruct((NS, EPT), jnp.int32),
        jax.ShapeDtypeStruct((NS, EPT), jnp.int32),
    ],
    mesh=_mesh,
    scratch_types=[
        pltpu.VMEM((EPT,), jnp.int32),
        pltpu.VMEM((EPT,), jnp.int32),
        pltpu.VMEM((EPT,), jnp.int32),
        pltpu.VMEM((NBKT, 16), jnp.int32),
        pltpu.VMEM((NBKT, 16), jnp.int32),
    ],
    compiler_params=pltpu.CompilerParams(
        use_tc_tiling_on_sc=False, needs_layout_passes=False),
)


def _sort_body(src_hbm, dst_hbm, outs_hbm, outd_hbm,
               srcv, dstv, sortv, hist, offs):
    c = lax.axis_index("c")
    s = lax.axis_index("s")
    lanes = lax.iota(jnp.int32, 16)
    ones = jnp.ones((16,), jnp.int32)

    pltpu.sync_copy(src_hbm.at[s], srcv)
    pltpu.sync_copy(dst_hbm.at[s], dstv)

    zero16 = jnp.zeros((16,), jnp.int32)

    def zh(i):
        hist[i] = zero16

    pl.loop(0, NBKT)(zh)

    def histpass(e):
        v = srcv[pl.ds(e * 16, 16)]
        plsc.addupdate_scatter(hist, [v >> 4, lanes], ones)

    pl.loop(0, EPL)(histpass)

    def prefix(b, acc):
        h = hist[b]
        incl = jnp.cumsum(h)
        offs[b] = acc + incl - h
        return acc + jnp.sum(h)

    pl.loop(0, NBKT, init_carry=jnp.int32(0))(prefix)

    def place(e):
        v = srcv[pl.ds(e * 16, 16)]
        b = v >> 4
        p = plsc.load_gather(offs, [b, lanes])
        plsc.store_scatter(offs, [b, lanes], p + 1)
        val = jnp.where(c == 0, v, dstv[pl.ds(e * 16, 16)])
        plsc.store_scatter(sortv, [p], val)

    pl.loop(0, EPL)(place)

    @pl.when(c == 0)
    def _():
        pltpu.sync_copy(sortv, outs_hbm.at[s])

    @pl.when(c == 1)
    def _():
        pltpu.sync_copy(sortv, outd_hbm.at[s])


_sort_kernel = pl.kernel(_sort_body, **_SORT_KWARGS)


_AGG_KWARGS = dict(
    out_type=jax.ShapeDtypeStruct((NC, NPAD, HH), jnp.float32),
    mesh=_mesh,
    scratch_types=[
        pltpu.VMEM((NCHUNK, CH), jnp.int32),
        pltpu.VMEM((NCHUNK, CH), jnp.int32),
        pltpu.VMEM((CH, HH), jnp.float32),
        pltpu.VMEM((CH, HH), jnp.float32),
        pltpu.VMEM((CH, HH), jnp.float32),
        pltpu.VMEM((CH, HH), jnp.float32),
        pltpu.VMEM((CHD, HH), jnp.float32),
        pltpu.VMEM_SHARED((NPAD, HH), jnp.float32),
        pltpu.SemaphoreType.DMA,
        pltpu.SemaphoreType.DMA,
        pltpu.SemaphoreType.DMA,
        pltpu.SemaphoreType.DMA,
        pltpu.SemaphoreType.DMA,
        pltpu.SemaphoreType.DMA,
        pltpu.SemaphoreType.DMA,
        pltpu.SemaphoreType.DMA,
    ],
    compiler_params=pltpu.CompilerParams(use_tc_tiling_on_sc=False),
)


def _agg_body(zh_hbm, src_hbm, dst_hbm, out_hbm,
              idxs, idxd, rows0, rows1, rows2, rows3, zb, acc,
              gs0, gs1, gs2, gs3, ss0, ss1, ss2, ss3):
    rows = (rows0, rows1, rows2, rows3)
    gs = (gs0, gs1, gs2, gs3)
    ss = (ss0, ss1, ss2, ss3)
    c = lax.axis_index("c")
    s = lax.axis_index("s")

    pltpu.sync_copy(src_hbm.at[s], idxs)
    pltpu.sync_copy(dst_hbm.at[s], idxd)

    def xform(i):
        for k in range(CH // 16):
            sl = pl.ds(k * 16, 16)
            idxs[i, sl] = idxs[i, sl] * 2 + c

    pl.loop(0, NCHUNK)(xform)

    _zero_rows(zb, CHD, HH)
    base = s * ROWS_PER_TILE
    for k in range(ROWS_PER_TILE // CHD):
        pltpu.sync_copy(zb, acc.at[pl.ds(base + k * CHD, CHD)])
    plsc.subcore_barrier()

    def gather(j, buf, sem):
        pltpu.async_copy(zh_hbm.at[idxs.at[j]], buf, sem)

    def wait_gather(j, buf, sem):
        pltpu.make_async_copy(zh_hbm.at[idxs.at[j]], buf, sem).wait()

    def scatter(j, buf, sem):
        pltpu.async_copy(buf, acc.at[idxd.at[j]], sem, add=True)

    def wait_scatter(j, buf, sem):
        pltpu.make_async_copy(buf, acc.at[idxd.at[j]], sem).wait()

    NB = 4
    for b in range(NB):
        gather(b, rows[b], gs[b])

    def body(t):
        j = NB * t
        for b in range(NB):
            wait_gather(j + b, rows[b], gs[b])
            scatter(j + b, rows[b], ss[b])

        @pl.when(t < NCHUNK // NB - 1)
        def _():
            for b in range(NB):
                wait_scatter(j + b, rows[b], ss[b])
                gather(j + NB + b, rows[b], gs[b])

    pl.loop(0, NCHUNK // NB)(body)

    jlast = NCHUNK - NB
    for b in range(NB):
        wait_scatter(jlast + b, rows[b], ss[b])
    plsc.subcore_barrier()

    for k in range(ROWS_PER_TILE // CHD):
        pltpu.sync_copy(acc.at[pl.ds(base + k * CHD, CHD)], zb)
        pltpu.sync_copy(zb, out_hbm.at[c, pl.ds(base + k * CHD, CHD)])


_agg_kernel = pl.kernel(_agg_body, **_AGG_KWARGS)


def _mm(a, b):
    return jnp.dot(a, b, preferred_element_type=jnp.float32)


def _enc_body(x_ref, pos_ref, degp_ref, wa_ref, wb_ref, encb_ref,
              convw_ref, resw_ref, resb_ref, z_ref, r_ref, dinv_ref):
    dinv = lax.rsqrt(1.0 + degp_ref[0] + degp_ref[1])
    xh = _mm(x_ref[...], wa_ref[...]) + _mm(pos_ref[...], wb_ref[...]) + encb_ref[...]
    z_ref[...] = dinv * _mm(xh, convw_ref[...])
    r_ref[...] = _mm(xh, resw_ref[...]) + resb_ref[...]
    dinv_ref[...] = dinv


def _agg_full(p_ref, zp_ref):
    return jnp.concatenate([p_ref[0], p_ref[1]], axis=-1) + zp_ref[...]


def _layer_body(p_ref, zp_ref, rp_ref, dinv_ref, convb_ref,
                convw_ref, resw_ref, resb_ref, z_ref, r_ref):
    dinv = dinv_ref[...]
    xh = jnp.maximum(
        dinv * _agg_full(p_ref, zp_ref) + convb_ref[...] + rp_ref[...], 0.0)
    z_ref[...] = dinv * _mm(xh, convw_ref[...])
    r_ref[...] = _mm(xh, resw_ref[...]) + resb_ref[...]


def _final_body(p_ref, zp_ref, rp_ref, dinv_ref, convb_ref,
                decw_ref, decb_ref, batch_ref, pool_ref):
    i = pl.program_id(0)
    xh = jnp.maximum(
        dinv_ref[...] * _agg_full(p_ref, zp_ref) + convb_ref[...] + rp_ref[...],
        0.0)
    outcol = jnp.sum(xh * decw_ref[...], axis=1, keepdims=True) + decb_ref[...]
    seg = lax.broadcasted_iota(jnp.int32, (NGRAPHS, BLK), 0)
    mask = (seg == batch_ref[...]).astype(jnp.float32)
    contrib = _mm(mask, outcol)

    @pl.when(i == 0)
    def _():
        pool_ref[...] = jnp.zeros_like(pool_ref)

    pool_ref[...] += contrib


def _rowb():
    return pl.BlockSpec((BLK, H), lambda i: (i, 0))


def _partb():
    return pl.BlockSpec((NC, BLK, HH), lambda i: (0, i, 0))


def _colb():
    return pl.BlockSpec((BLK, 1), lambda i: (i, 0))


def _wb(r):
    return pl.BlockSpec((r, H), lambda i: (0, 0))


_enc_call = pl.pallas_call(
    _enc_body,
    grid=(GRID,),
    in_specs=[
        _rowb(),
        pl.BlockSpec((BLK, 8), lambda i: (i, 0)),
        pl.BlockSpec((NC, BLK, 1), lambda i: (0, i, 0)),
        _wb(H), pl.BlockSpec((8, H), lambda i: (0, 0)),
        _wb(1),
        _wb(H), _wb(H), _wb(1),
    ],
    out_specs=[_rowb(), _rowb(), _colb()],
    out_shape=[
        jax.ShapeDtypeStruct((NPAD, H), jnp.float32),
        jax.ShapeDtypeStruct((NPAD, H), jnp.float32),
        jax.ShapeDtypeStruct((NPAD, 1), jnp.float32),
    ],
)

_layer_call = pl.pallas_call(
    _layer_body,
    grid=(GRID,),
    in_specs=[
        _partb(),
        _rowb(), _rowb(),
        _colb(),
        _wb(1),
        _wb(H), _wb(H), _wb(1),
    ],
    out_specs=[_rowb(), _rowb()],
    out_shape=[
        jax.ShapeDtypeStruct((NPAD, H), jnp.float32),
        jax.ShapeDtypeStruct((NPAD, H), jnp.float32),
    ],
)

_final_call = pl.pallas_call(
    _final_body,
    grid=(GRID,),
    in_specs=[
        _partb(),
        _rowb(), _rowb(),
        _colb(),
        _wb(1),
        _wb(1),
        pl.BlockSpec((1, 1), lambda i: (0, 0)),
        pl.BlockSpec((1, BLK), lambda i: (0, i)),
    ],
    out_specs=pl.BlockSpec((NGRAPHS, 1), lambda i: (0, 0)),
    out_shape=jax.ShapeDtypeStruct((NGRAPHS, 1), jnp.float32),
)


def kernel(x, pos, edge_index, batch, enc_W, enc_b, conv_W, conv_b,
           res_W, res_b, dec_W, dec_b):
    src = edge_index[0]
    dst = edge_index[1]
    padlen = EPAD - E
    src_t = jnp.concatenate([src, jnp.full((padlen,), N, jnp.int32)]).reshape(
        NS, NCHUNK, CH)
    dst_t = jnp.concatenate([dst, jnp.full((padlen,), N, jnp.int32)]).reshape(
        NS, NCHUNK, CH)

    x_p = jnp.pad(x, ((0, NPAD - N), (0, 0)))
    pos_p = jnp.pad(pos, ((0, NPAD - N), (0, 8 - pos.shape[1])))
    batch_row = jnp.pad(batch, (0, NPAD - N), constant_values=NGRAPHS).reshape(
        1, NPAD)

    wa = enc_W[:H]
    wb = jnp.pad(enc_W[H:], ((0, 8 - (enc_W.shape[0] - H)), (0, 0)))
    encb = enc_b.reshape(1, H)
    convb = conv_b.reshape(1, H)
    resb = res_b.reshape(1, H)
    decw = dec_W.reshape(1, H)
    decb = dec_b.reshape(1, 1)

    src_s, dst_s = _sort_kernel(src_t.reshape(NS, EPT), dst_t.reshape(NS, EPT))
    src_t = src_s.reshape(NS, NCHUNK, CH)
    dst_t = dst_s.reshape(NS, NCHUNK, CH)
    degp = _deg_kernel(dst_t.reshape(NS, NCHUNKD, CHD))
    degp = degp[:, :, :1]

    z, r, dinv = _enc_call(x_p, pos_p, degp, wa, wb, encb, conv_W, res_W, resb)

    for _ in range(2):
        p = _agg_kernel(z.reshape(2 * NPAD, HH), src_t, dst_t)
        z, r = _layer_call(p, z, r, dinv, convb, conv_W, res_W, resb)
    p = _agg_kernel(z.reshape(2 * NPAD, HH), src_t, dst_t)

    pooled = _final_call(p, z, r, dinv, convb, decw, decb, batch_row)
    return pooled.reshape(NGRAPHS)


# speedup vs baseline: 1.1104x; 1.1104x over previous
import functools

import jax
import jax.numpy as jnp
from jax import lax
from jax.experimental import pallas as pl
from jax.experimental.pallas import tpu as pltpu
from jax.experimental.pallas import tpu_sc as plsc

N = 10000
E = 320000
H = 128
HH = H // 2
NGRAPHS = 64

NPAD = 10240
NC, NS = 2, 16
CH = 128
NCHUNK = 160
CHD = 128
NCHUNKD = 160
EPAD = NS * NCHUNK * CH
ROWS_PER_TILE = NPAD // NS
BLK = 1280
GRID = NPAD // BLK

_mesh = plsc.VectorSubcoreMesh(core_axis_name="c", subcore_axis_name="s")


def _zero_rows(ref, nrows, ncols):
    zero16 = jnp.zeros((16,), jnp.float32)

    def body(i):
        for k in range(ncols // 16):
            ref[i, pl.ds(k * 16, 16)] = zero16

    pl.loop(0, nrows)(body)


_DEG_KWARGS = dict(
    out_type=jax.ShapeDtypeStruct((NC, NPAD, 16), jnp.float32),
    mesh=_mesh,
    scratch_types=[
        pltpu.VMEM((NCHUNKD // 2, CHD), jnp.int32),
        pltpu.VMEM((CHD, 16), jnp.float32),
        pltpu.VMEM((CHD, 16), jnp.float32),
        pltpu.VMEM_SHARED((NPAD, 16), jnp.float32),
        pltpu.SemaphoreType.DMA,
    ],
    compiler_params=pltpu.CompilerParams(use_tc_tiling_on_sc=False),
)


def _deg_body(dst_hbm, out_hbm, idxd, ones_r, zb, acc, sem):
    c = lax.axis_index("c")
    s = lax.axis_index("s")

    pltpu.sync_copy(dst_hbm.at[s, pl.ds(c * (NCHUNKD // 2), NCHUNKD // 2)], idxd)

    one0 = jnp.where(lax.iota(jnp.int32, 16) == 0, 1.0, 0.0).astype(jnp.float32)
    zero16 = jnp.zeros((16,), jnp.float32)

    def init(i):
        ones_r[i] = one0
        zb[i] = zero16

    pl.loop(0, CHD)(init)

    base = s * ROWS_PER_TILE
    for k in range(ROWS_PER_TILE // CHD):
        pltpu.sync_copy(zb, acc.at[pl.ds(base + k * CHD, CHD)])
    plsc.subcore_barrier()

    def fire(j):
        descs = [
            pltpu.async_copy(ones_r, acc.at[idxd.at[j + k]], sem, add=True)
            for k in range(8)
        ]
        for d in descs:
            d.wait()

    pl.loop(0, NCHUNKD // 2, step=8)(fire)
    plsc.subcore_barrier()

    for k in range(ROWS_PER_TILE // CHD):
        pltpu.sync_copy(acc.at[pl.ds(base + k * CHD, CHD)], zb)
        pltpu.sync_copy(zb, out_hbm.at[c, pl.ds(base + k * CHD, CHD)])


_deg_kernel = pl.kernel(_deg_body, **_DEG_KWARGS)


NBKT = 640
EPT = NCHUNK * CH
EPL = EPT // 16

_SORT_KWARGS = dict(
    out_type=[
        jax.ShapeDtypeStruct((NS, EPT), jnp.int32),
        jax.ShapeDtypeStruct((NS, EPT), jnp.int32),
    ],
    mesh=_mesh,
    scratch_types=[
        pltpu.VMEM((EPT,), jnp.int32),
        pltpu.VMEM((EPT,), jnp.int32),
        pltpu.VMEM((EPT,), jnp.int32),
        pltpu.VMEM((NBKT, 16), jnp.int32),
        pltpu.VMEM((NBKT, 16), jnp.int32),
    ],
    compiler_params=pltpu.CompilerParams(
        use_tc_tiling_on_sc=False, needs_layout_passes=False),
)


def _sort_body(src_hbm, dst_hbm, outs_hbm, outd_hbm,
               srcv, dstv, sortv, hist, offs):
    c = lax.axis_index("c")
    s = lax.axis_index("s")
    lanes = lax.iota(jnp.int32, 16)
    ones = jnp.ones((16,), jnp.int32)

    pltpu.sync_copy(src_hbm.at[s], srcv)
    pltpu.sync_copy(dst_hbm.at[s], dstv)

    zero16 = jnp.zeros((16,), jnp.int32)

    def zh(i):
        hist[i] = zero16

    pl.loop(0, NBKT)(zh)

    def histpass(e):
        v = srcv[pl.ds(e * 16, 16)]
        plsc.addupdate_scatter(hist, [v >> 4, lanes], ones)

    pl.loop(0, EPL)(histpass)

    def prefix(b, acc):
        h = hist[b]
        incl = jnp.cumsum(h)
        offs[b] = acc + incl - h
        return acc + jnp.sum(h)

    pl.loop(0, NBKT, init_carry=jnp.int32(0))(prefix)

    def place(e):
        v = srcv[pl.ds(e * 16, 16)]
        b = v >> 4
        p = plsc.load_gather(offs, [b, lanes])
        plsc.store_scatter(offs, [b, lanes], p + 1)
        val = jnp.where(c == 0, v, dstv[pl.ds(e * 16, 16)])
        plsc.store_scatter(sortv, [p], val)

    pl.loop(0, EPL)(place)

    @pl.when(c == 0)
    def _():
        pltpu.sync_copy(sortv, outs_hbm.at[s])

    @pl.when(c == 1)
    def _():
        pltpu.sync_copy(sortv, outd_hbm.at[s])


_sort_kernel = pl.kernel(_sort_body, **_SORT_KWARGS)


_AGG_KWARGS = dict(
    out_type=jax.ShapeDtypeStruct((NC, NPAD, HH), jnp.float32),
    mesh=_mesh,
    scratch_types=[
        pltpu.VMEM((NCHUNK, CH), jnp.int32),
        pltpu.VMEM((NCHUNK, CH), jnp.int32),
        pltpu.VMEM((CH, HH), jnp.float32),
        pltpu.VMEM((CH, HH), jnp.float32),
        pltpu.VMEM((CH, HH), jnp.float32),
        pltpu.VMEM((CH, HH), jnp.float32),
        pltpu.VMEM((CHD, HH), jnp.float32),
        pltpu.VMEM_SHARED((NPAD, HH), jnp.float32),
        pltpu.SemaphoreType.DMA,
        pltpu.SemaphoreType.DMA,
        pltpu.SemaphoreType.DMA,
        pltpu.SemaphoreType.DMA,
        pltpu.SemaphoreType.DMA,
        pltpu.SemaphoreType.DMA,
        pltpu.SemaphoreType.DMA,
        pltpu.SemaphoreType.DMA,
    ],
    compiler_params=pltpu.CompilerParams(use_tc_tiling_on_sc=False),
)


def _agg_body(zh_hbm, src_hbm, dst_hbm, out_hbm,
              idxs, idxd, rows0, rows1, rows2, rows3, zb, acc,
              gs0, gs1, gs2, gs3, ss0, ss1, ss2, ss3):
    rows = (rows0, rows1, rows2, rows3)
    gs = (gs0, gs1, gs2, gs3)
    ss = (ss0, ss1, ss2, ss3)
    c = lax.axis_index("c")
    s = lax.axis_index("s")

    pltpu.sync_copy(src_hbm.at[s], idxs)
    pltpu.sync_copy(dst_hbm.at[s], idxd)

    _zero_rows(zb, CHD, HH)
    base = s * ROWS_PER_TILE
    for k in range(ROWS_PER_TILE // CHD):
        pltpu.sync_copy(zb, acc.at[pl.ds(base + k * CHD, CHD)])
    plsc.subcore_barrier()

    def gather(j, buf, sem):
        pltpu.async_copy(zh_hbm.at[c].at[idxs.at[j]], buf, sem)

    def wait_gather(j, buf, sem):
        pltpu.make_async_copy(zh_hbm.at[c].at[idxs.at[j]], buf, sem).wait()

    def scatter(j, buf, sem):
        pltpu.async_copy(buf, acc.at[idxd.at[j]], sem, add=True)

    def wait_scatter(j, buf, sem):
        pltpu.make_async_copy(buf, acc.at[idxd.at[j]], sem).wait()

    NB = 4
    for b in range(NB):
        gather(b, rows[b], gs[b])

    def body(t):
        j = NB * t
        for b in range(NB):
            wait_gather(j + b, rows[b], gs[b])
            scatter(j + b, rows[b], ss[b])

        @pl.when(t < NCHUNK // NB - 1)
        def _():
            for b in range(NB):
                wait_scatter(j + b, rows[b], ss[b])
                gather(j + NB + b, rows[b], gs[b])

    pl.loop(0, NCHUNK // NB)(body)

    jlast = NCHUNK - NB
    for b in range(NB):
        wait_scatter(jlast + b, rows[b], ss[b])
    plsc.subcore_barrier()

    for k in range(ROWS_PER_TILE // CHD):
        pltpu.sync_copy(acc.at[pl.ds(base + k * CHD, CHD)], zb)
        pltpu.sync_copy(zb, out_hbm.at[c, pl.ds(base + k * CHD, CHD)])


_agg_kernel = pl.kernel(_agg_body, **_AGG_KWARGS)


def _mm(a, b):
    return jnp.dot(a, b, preferred_element_type=jnp.float32)


def _enc_body(x_ref, pos_ref, degp_ref, wa_ref, wb_ref, encb_ref,
              convw_ref, resw_ref, resb_ref, z_ref, r_ref, dinv_ref):
    dinv = lax.rsqrt(1.0 + degp_ref[0] + degp_ref[1])
    xh = _mm(x_ref[...], wa_ref[...]) + _mm(pos_ref[...], wb_ref[...]) + encb_ref[...]
    z = dinv * _mm(xh, convw_ref[...])
    z_ref[0] = z[:, :HH]
    z_ref[1] = z[:, HH:]
    r_ref[...] = _mm(xh, resw_ref[...]) + resb_ref[...]
    dinv_ref[...] = dinv


def _agg_full(p_ref, zp_ref):
    return (jnp.concatenate([p_ref[0], p_ref[1]], axis=-1)
            + jnp.concatenate([zp_ref[0], zp_ref[1]], axis=-1))


def _layer_body(p_ref, zp_ref, rp_ref, dinv_ref, convb_ref,
                convw_ref, resw_ref, resb_ref, z_ref, r_ref):
    dinv = dinv_ref[...]
    xh = jnp.maximum(
        dinv * _agg_full(p_ref, zp_ref) + convb_ref[...] + rp_ref[...], 0.0)
    z = dinv * _mm(xh, convw_ref[...])
    z_ref[0] = z[:, :HH]
    z_ref[1] = z[:, HH:]
    r_ref[...] = _mm(xh, resw_ref[...]) + resb_ref[...]


def _final_body(p_ref, zp_ref, rp_ref, dinv_ref, convb_ref,
                decw_ref, decb_ref, batch_ref, pool_ref):
    i = pl.program_id(0)
    xh = jnp.maximum(
        dinv_ref[...] * _agg_full(p_ref, zp_ref) + convb_ref[...] + rp_ref[...],
        0.0)
    outcol = jnp.sum(xh * decw_ref[...], axis=1, keepdims=True) + decb_ref[...]
    seg = lax.broadcasted_iota(jnp.int32, (NGRAPHS, BLK), 0)
    mask = (seg == batch_ref[...]).astype(jnp.float32)
    contrib = _mm(mask, outcol)

    @pl.when(i == 0)
    def _():
        pool_ref[...] = jnp.zeros_like(pool_ref)

    pool_ref[...] += contrib


def _rowb():
    return pl.BlockSpec((BLK, H), lambda i: (i, 0))


def _partb():
    return pl.BlockSpec((NC, BLK, HH), lambda i: (0, i, 0))


def _colb():
    return pl.BlockSpec((BLK, 1), lambda i: (i, 0))


def _wb(r):
    return pl.BlockSpec((r, H), lambda i: (0, 0))


_enc_call = pl.pallas_call(
    _enc_body,
    grid=(GRID,),
    in_specs=[
        _rowb(),
        pl.BlockSpec((BLK, 8), lambda i: (i, 0)),
        pl.BlockSpec((NC, BLK, 1), lambda i: (0, i, 0)),
        _wb(H), pl.BlockSpec((8, H), lambda i: (0, 0)),
        _wb(1),
        _wb(H), _wb(H), _wb(1),
    ],
    out_specs=[_partb(), _rowb(), _colb()],
    out_shape=[
        jax.ShapeDtypeStruct((NC, NPAD, HH), jnp.float32),
        jax.ShapeDtypeStruct((NPAD, H), jnp.float32),
        jax.ShapeDtypeStruct((NPAD, 1), jnp.float32),
    ],
)

_layer_call = pl.pallas_call(
    _layer_body,
    grid=(GRID,),
    in_specs=[
        _partb(),
        _partb(), _rowb(),
        _colb(),
        _wb(1),
        _wb(H), _wb(H), _wb(1),
    ],
    out_specs=[_partb(), _rowb()],
    out_shape=[
        jax.ShapeDtypeStruct((NC, NPAD, HH), jnp.float32),
        jax.ShapeDtypeStruct((NPAD, H), jnp.float32),
    ],
)

_final_call = pl.pallas_call(
    _final_body,
    grid=(GRID,),
    in_specs=[
        _partb(),
        _partb(), _rowb(),
        _colb(),
        _wb(1),
        _wb(1),
        pl.BlockSpec((1, 1), lambda i: (0, 0)),
        pl.BlockSpec((1, BLK), lambda i: (0, i)),
    ],
    out_specs=pl.BlockSpec((NGRAPHS, 1), lambda i: (0, 0)),
    out_shape=jax.ShapeDtypeStruct((NGRAPHS, 1), jnp.float32),
)


def kernel(x, pos, edge_index, batch, enc_W, enc_b, conv_W, conv_b,
           res_W, res_b, dec_W, dec_b):
    src = edge_index[0]
    dst = edge_index[1]
    padlen = EPAD - E
    src_t = jnp.concatenate([src, jnp.full((padlen,), N, jnp.int32)]).reshape(
        NS, NCHUNK, CH)
    dst_t = jnp.concatenate([dst, jnp.full((padlen,), N, jnp.int32)]).reshape(
        NS, NCHUNK, CH)

    x_p = jnp.pad(x, ((0, NPAD - N), (0, 0)))
    pos_p = jnp.pad(pos, ((0, NPAD - N), (0, 8 - pos.shape[1])))
    batch_row = jnp.pad(batch, (0, NPAD - N), constant_values=NGRAPHS).reshape(
        1, NPAD)

    wa = enc_W[:H]
    wb = jnp.pad(enc_W[H:], ((0, 8 - (enc_W.shape[0] - H)), (0, 0)))
    encb = enc_b.reshape(1, H)
    convb = conv_b.reshape(1, H)
    resb = res_b.reshape(1, H)
    decw = dec_W.reshape(1, H)
    decb = dec_b.reshape(1, 1)

    src_s, dst_s = _sort_kernel(src_t.reshape(NS, EPT), dst_t.reshape(NS, EPT))
    src_t = src_s.reshape(NS, NCHUNK, CH)
    dst_t = dst_s.reshape(NS, NCHUNK, CH)
    degp = _deg_kernel(dst_t.reshape(NS, NCHUNKD, CHD))
    degp = degp[:, :, :1]

    z, r, dinv = _enc_call(x_p, pos_p, degp, wa, wb, encb, conv_W, res_W, resb)

    for _ in range(2):
        p = _agg_kernel(z, src_t, dst_t)
        z, r = _layer_call(p, z, r, dinv, convb, conv_W, res_W, resb)
    p = _agg_kernel(z, src_t, dst_t)

    pooled = _final_call(p, z, r, dinv, convb, decw, decb, batch_row)
    return pooled.reshape(NGRAPHS)
